# baseline (speedup 1.0000x reference)
import jax
import jax.numpy as jnp
from jax.experimental import pallas as pl


def _iota(shape, dim):
    return jax.lax.broadcasted_iota(jnp.int32, shape, dim)


def _body(x_ref, g_ref, o_ref):
    xb = x_ref[0]
    g = g_ref[0]
    rows = xb.shape[0]
    tb, e = g.shape
    grp = rows // tb
    mx = jnp.max(g, axis=-1, keepdims=True)
    eq = (g == mx).astype(jnp.float32)
    tri = (_iota((e, e), 0) <= _iota((e, e), 1)).astype(jnp.float32)
    cs = jax.lax.dot(eq, tri, preferred_element_type=jnp.float32)
    oh = eq * (cs == 1.0).astype(jnp.float32)
    tmat = (_iota((e, 128), 1) % e == _iota((e, 128), 0)).astype(jnp.float32)
    m128 = jax.lax.dot(oh, tmat, preferred_element_type=jnp.float32)
    expand = (_iota((rows, tb), 0) // grp == _iota((rows, tb), 1)).astype(jnp.float32)
    mrows = jax.lax.dot(expand, m128, preferred_element_type=jnp.float32)
    y = xb * mrows
    s = (_iota((128, 16), 0) // e == _iota((128, 16), 1)).astype(jnp.float32)
    o_ref[0] = jax.lax.dot(y, s, preferred_element_type=jnp.float32)


def kernel(experts, gate):
    b, t, d, e = experts.shape
    grp = d // 16
    x = experts.reshape(b, t * grp, 128)
    tb = 32
    grid = (b, t // tb)
    out = pl.pallas_call(
        _body,
        grid=grid,
        in_specs=[
            pl.BlockSpec((1, tb * grp, 128), lambda i, j: (i, j, 0)),
            pl.BlockSpec((1, tb, e), lambda i, j: (i, j, 0)),
        ],
        out_specs=pl.BlockSpec((1, tb * grp, 16), lambda i, j: (i, j, 0)),
        out_shape=jax.ShapeDtypeStruct((b, t * grp, 16), jnp.float32),
    )(x, gate)
    return out.reshape(b, t, d)


# speedup vs baseline: 35.0759x; 35.0759x over previous
import jax
import jax.numpy as jnp
from jax.experimental import pallas as pl


def _iota(shape, dim):
    return jax.lax.broadcasted_iota(jnp.int32, shape, dim)


def _body(x_ref, g_ref, o_ref):
    xb = x_ref[0]
    gt = g_ref[0]
    tb, e, d = xb.shape
    mx = jnp.max(gt, axis=0, keepdims=True)
    eq = (gt == mx).astype(jnp.float32)
    lower = (_iota((e, e), 1) <= _iota((e, e), 0)).astype(jnp.float32)
    cs = jax.lax.dot(lower, eq, preferred_element_type=jnp.float32)
    oht = eq * (cs == 1.0).astype(jnp.float32)
    oh = oht.T
    xr = xb.reshape(tb * e, d)
    tile8 = (_iota((e, tb * e), 1) % e == _iota((e, tb * e), 0)).astype(jnp.float32)
    ohtile = jax.lax.dot(oh, tile8, preferred_element_type=jnp.float32)
    blockdiag = (_iota((tb, tb * e), 1) // e == _iota((tb, tb * e), 0))
    sel = jnp.where(blockdiag, ohtile, 0.0)
    o_ref[0] = jax.lax.dot(sel, xr, preferred_element_type=jnp.float32)


def kernel(experts, gate):
    b, t, d, e = experts.shape
    xt = experts.transpose(0, 1, 3, 2)
    gt = gate.transpose(0, 2, 1)
    tb = 128
    grid = (b, t // tb)
    return pl.pallas_call(
        _body,
        grid=grid,
        in_specs=[
            pl.BlockSpec((1, tb, e, d), lambda i, j: (i, j, 0, 0)),
            pl.BlockSpec((1, e, tb), lambda i, j: (i, 0, j)),
        ],
        out_specs=pl.BlockSpec((1, tb, d), lambda i, j: (i, j, 0)),
        out_shape=jax.ShapeDtypeStruct((b, t, d), jnp.float32),
    )(xt, gt)


# speedup vs baseline: 83.7431x; 2.3875x over previous
import functools

import jax
import jax.numpy as jnp
from jax import lax
from jax.experimental import pallas as pl
from jax.experimental.pallas import tpu as pltpu, tpu_sc as plsc

_NW = 32
_CH = 128


def _vgather16(x, idx):
    dn = lax.GatherDimensionNumbers(
        offset_dims=(), collapsed_slice_dims=(0,), start_index_map=(0,))
    return lax.gather(x, idx[:, None], dn, (1,),
                      mode=lax.GatherScatterMode.PROMISE_IN_BOUNDS)


def _sc_body(t_total, z2, gt, out, idx_v, data_v, gate_v, gsem, osem):
    nc = 2
    wid = lax.axis_index("s") * nc + lax.axis_index("c")
    rows_per_w = (t_total * 4 * 8) // _NW
    units_per_w = rows_per_w // 64
    u0 = wid * units_per_w
    m = u0 // (t_total // 8)
    ts = (u0 % (t_total // 8)) * 8
    pltpu.sync_copy(gt.at[m, :, pl.ds(ts, units_per_w * 8)], gate_v)
    it16 = lax.broadcasted_iota(jnp.int32, (16,), 0)
    pat_lo = lax.bitwise_and(it16, 7)
    pat_hi = pat_lo + 8
    jpat = lax.shift_right_logical(it16, 3) * 8
    niter = rows_per_w // _CH

    def fire_gather(i):
        buf = i % 2
        toff = i * 16
        best = gate_v[0, pl.ds(toff, 16)]
        besti = jnp.zeros((16,), jnp.int32)
        for ee in range(1, 8):
            ge = gate_v[ee, pl.ds(toff, 16)]
            gtr = ge > best
            besti = jnp.where(gtr, ee, besti)
            best = jnp.where(gtr, ge, best)
        v16 = m * (t_total * 64) + (ts + toff + it16) * 64 + besti
        dup_lo = _vgather16(v16, pat_lo)
        dup_hi = _vgather16(v16, pat_hi)
        for k in range(8):
            vdup = dup_lo if k < 4 else dup_hi
            idx_v[buf, pl.ds(16 * k, 16)] = vdup + (16 * (k % 4) + jpat)
        pltpu.async_copy(z2.at[idx_v.at[buf]], data_v.at[buf], gsem)

    def wait_gather(i):
        buf = i % 2
        pltpu.make_async_copy(z2.at[idx_v.at[buf]], data_v.at[buf], gsem).wait()

    def fire_out(i):
        qb = wid * rows_per_w + i * _CH
        pltpu.async_copy(data_v.at[i % 2], out.at[pl.ds(qb, _CH)], osem)

    def wait_out(i):
        qb = wid * rows_per_w + i * _CH
        pltpu.make_async_copy(data_v.at[i % 2], out.at[pl.ds(qb, _CH)], osem).wait()

    fire_gather(0)
    for i in range(niter):
        if i + 1 < niter:
            if i >= 1:
                wait_out(i - 1)
            fire_gather(i + 1)
        wait_gather(i)
        fire_out(i)
    wait_out(niter - 2)
    wait_out(niter - 1)


def kernel(experts, gate):
    b, t, d, e = experts.shape
    nj = d // 128
    z2 = (experts.transpose(0, 1, 3, 2)
          .reshape(b, t, e, nj, 128)
          .transpose(0, 1, 3, 2, 4)
          .reshape(b * t * nj * e, 128))
    gt = gate.transpose(0, 2, 1)
    mesh = plsc.VectorSubcoreMesh(core_axis_name="c", subcore_axis_name="s")
    run = functools.partial(
        pl.kernel,
        mesh=mesh,
        out_type=jax.ShapeDtypeStruct((b * t * nj, 128), jnp.float32),
        scratch_types=[
            pltpu.VMEM((2, _CH), jnp.int32),
            pltpu.VMEM((2, _CH, 128), jnp.float32),
            pltpu.VMEM((8, 256), jnp.float32),
            pltpu.SemaphoreType.DMA,
            pltpu.SemaphoreType.DMA,
        ],
    )(functools.partial(_sc_body, t))
    out2 = run(z2, gt)
    return (out2.reshape(b, t // 8, nj, 8, 128)
            .transpose(0, 1, 3, 2, 4)
            .reshape(b, t, d))


# speedup vs baseline: 85.4483x; 1.0204x over previous
import functools

import jax
import jax.numpy as jnp
from jax import lax
from jax.experimental import pallas as pl
from jax.experimental.pallas import tpu as pltpu, tpu_sc as plsc

_NW = 32
_CH = 128
_NB = 4


def _vgather16(x, idx):
    dn = lax.GatherDimensionNumbers(
        offset_dims=(), collapsed_slice_dims=(0,), start_index_map=(0,))
    return lax.gather(x, idx[:, None], dn, (1,),
                      mode=lax.GatherScatterMode.PROMISE_IN_BOUNDS)


def _sc_body(t_total, z2, gt, out, idx_v, data_v, gate_v, gsem, osem):
    nc = 2
    wid = lax.axis_index("s") * nc + lax.axis_index("c")
    rows_per_w = (t_total * 4 * 8) // _NW
    units_per_w = rows_per_w // 64
    u0 = wid * units_per_w
    m = u0 // (t_total // 8)
    ts = (u0 % (t_total // 8)) * 8
    pltpu.sync_copy(gt.at[m, :, pl.ds(ts, units_per_w * 8)], gate_v)
    it16 = lax.broadcasted_iota(jnp.int32, (16,), 0)
    pat_lo = lax.bitwise_and(it16, 7)
    pat_hi = pat_lo + 8
    jpat = lax.shift_right_logical(it16, 3) * 8
    niter = rows_per_w // _CH

    def fire_gather(i):
        buf = i % _NB
        toff = i * 16
        best = gate_v[0, pl.ds(toff, 16)]
        besti = jnp.zeros((16,), jnp.int32)
        for ee in range(1, 8):
            ge = gate_v[ee, pl.ds(toff, 16)]
            gtr = ge > best
            besti = jnp.where(gtr, ee, besti)
            best = jnp.where(gtr, ge, best)
        v16 = m * (t_total * 64) + (ts + toff + it16) * 64 + besti
        dup_lo = _vgather16(v16, pat_lo)
        dup_hi = _vgather16(v16, pat_hi)
        for k in range(8):
            vdup = dup_lo if k < 4 else dup_hi
            idx_v[buf, pl.ds(16 * k, 16)] = vdup + (16 * (k % 4) + jpat)
        pltpu.async_copy(z2.at[idx_v.at[buf]], data_v.at[buf], gsem)

    def wait_gather(i):
        buf = i % _NB
        pltpu.make_async_copy(z2.at[idx_v.at[buf]], data_v.at[buf], gsem).wait()

    def fire_out(i):
        qb = wid * rows_per_w + i * _CH
        pltpu.async_copy(data_v.at[i % _NB], out.at[pl.ds(qb, _CH)], osem)

    def wait_out(i):
        qb = wid * rows_per_w + i * _CH
        pltpu.make_async_copy(data_v.at[i % _NB], out.at[pl.ds(qb, _CH)], osem).wait()

    for p in range(_NB - 1):
        fire_gather(p)
    for i in range(niter):
        if i + _NB - 1 < niter:
            if i >= 1:
                wait_out(i - 1)
            fire_gather(i + _NB - 1)
        wait_gather(i)
        fire_out(i)
    for i in range(max(0, niter - _NB), niter):
        if i >= 1 or niter <= _NB:
            wait_out(i)


def kernel(experts, gate):
    b, t, d, e = experts.shape
    nj = d // 128
    z2 = (experts.transpose(0, 1, 3, 2)
          .reshape(b, t, e, nj, 128)
          .transpose(0, 1, 3, 2, 4)
          .reshape(b * t * nj * e, 128))
    gt = gate.transpose(0, 2, 1)
    mesh = plsc.VectorSubcoreMesh(core_axis_name="c", subcore_axis_name="s")
    run = functools.partial(
        pl.kernel,
        mesh=mesh,
        out_type=jax.ShapeDtypeStruct((b * t * nj, 128), jnp.float32),
        scratch_types=[
            pltpu.VMEM((_NB, _CH), jnp.int32),
            pltpu.VMEM((_NB, _CH, 128), jnp.float32),
            pltpu.VMEM((8, 256), jnp.float32),
            pltpu.SemaphoreType.DMA,
            pltpu.SemaphoreType.DMA,
        ],
    )(functools.partial(_sc_body, t))
    out2 = run(z2, gt)
    return (out2.reshape(b, t // 8, nj, 8, 128)
            .transpose(0, 1, 3, 2, 4)
            .reshape(b, t, d))


# speedup vs baseline: 85.5959x; 1.0017x over previous
import functools

import jax
import jax.numpy as jnp
from jax import lax
from jax.experimental import pallas as pl
from jax.experimental.pallas import tpu as pltpu, tpu_sc as plsc

_NW = 32
_CH = 128
_NB = 6


def _vgather16(x, idx):
    dn = lax.GatherDimensionNumbers(
        offset_dims=(), collapsed_slice_dims=(0,), start_index_map=(0,))
    return lax.gather(x, idx[:, None], dn, (1,),
                      mode=lax.GatherScatterMode.PROMISE_IN_BOUNDS)


def _sc_body(t_total, z2, gt, out, idx_v, data_v, gate_v, gsem, osem):
    nc = 2
    wid = lax.axis_index("s") * nc + lax.axis_index("c")
    rows_per_w = (t_total * 4 * 8) // _NW
    units_per_w = rows_per_w // 64
    u0 = wid * units_per_w
    m = u0 // (t_total // 8)
    ts = (u0 % (t_total // 8)) * 8
    pltpu.sync_copy(gt.at[m, :, pl.ds(ts, units_per_w * 8)], gate_v)
    it16 = lax.broadcasted_iota(jnp.int32, (16,), 0)
    pat_lo = lax.bitwise_and(it16, 7)
    pat_hi = pat_lo + 8
    jpat = lax.shift_right_logical(it16, 3) * 8
    niter = rows_per_w // _CH

    def fire_gather(i):
        buf = i % _NB
        toff = i * 16
        best = gate_v[0, pl.ds(toff, 16)]
        besti = jnp.zeros((16,), jnp.int32)
        for ee in range(1, 8):
            ge = gate_v[ee, pl.ds(toff, 16)]
            gtr = ge > best
            besti = jnp.where(gtr, ee, besti)
            best = jnp.where(gtr, ge, best)
        v16 = m * (t_total * 64) + (ts + toff + it16) * 64 + besti
        dup_lo = _vgather16(v16, pat_lo)
        dup_hi = _vgather16(v16, pat_hi)
        for k in range(8):
            vdup = dup_lo if k < 4 else dup_hi
            idx_v[buf, pl.ds(16 * k, 16)] = vdup + (16 * (k % 4) + jpat)
        pltpu.async_copy(z2.at[idx_v.at[buf]], data_v.at[buf], gsem)

    def wait_gather(i):
        buf = i % _NB
        pltpu.make_async_copy(z2.at[idx_v.at[buf]], data_v.at[buf], gsem).wait()

    def fire_out(i):
        qb = wid * rows_per_w + i * _CH
        pltpu.async_copy(data_v.at[i % _NB], out.at[pl.ds(qb, _CH)], osem)

    def wait_out(i):
        qb = wid * rows_per_w + i * _CH
        pltpu.make_async_copy(data_v.at[i % _NB], out.at[pl.ds(qb, _CH)], osem).wait()

    for p in range(_NB - 1):
        fire_gather(p)
    for i in range(niter):
        if i + _NB - 1 < niter:
            if i >= 1:
                wait_out(i - 1)
            fire_gather(i + _NB - 1)
        wait_gather(i)
        fire_out(i)
    for i in range(max(0, niter - _NB), niter):
        if i >= 1 or niter <= _NB:
            wait_out(i)


def kernel(experts, gate):
    b, t, d, e = experts.shape
    nj = d // 128
    z2 = (experts.transpose(0, 1, 3, 2)
          .reshape(b, t, e, nj, 128)
          .transpose(0, 1, 3, 2, 4)
          .reshape(b * t * nj * e, 128))
    gt = gate.transpose(0, 2, 1)
    mesh = plsc.VectorSubcoreMesh(core_axis_name="c", subcore_axis_name="s")
    run = functools.partial(
        pl.kernel,
        mesh=mesh,
        out_type=jax.ShapeDtypeStruct((b * t * nj, 128), jnp.float32),
        scratch_types=[
            pltpu.VMEM((_NB, _CH), jnp.int32),
            pltpu.VMEM((_NB, _CH, 128), jnp.float32),
            pltpu.VMEM((8, 256), jnp.float32),
            pltpu.SemaphoreType.DMA,
            pltpu.SemaphoreType.DMA,
        ],
    )(functools.partial(_sc_body, t))
    out2 = run(z2, gt)
    return (out2.reshape(b, t // 8, nj, 8, 128)
            .transpose(0, 1, 3, 2, 4)
            .reshape(b, t, d))


# speedup vs baseline: 86.0051x; 1.0048x over previous
import functools

import jax
import jax.numpy as jnp
from jax import lax
from jax.experimental import pallas as pl
from jax.experimental.pallas import tpu as pltpu, tpu_sc as plsc

_NW = 32
_CH = 128
_NB = 6


def _vgather16(x, idx):
    dn = lax.GatherDimensionNumbers(
        offset_dims=(), collapsed_slice_dims=(0,), start_index_map=(0,))
    return lax.gather(x, idx[:, None], dn, (1,),
                      mode=lax.GatherScatterMode.PROMISE_IN_BOUNDS)


def _sc_body(t_total, z2, gt, out, idx_v, data_v, gate_v, gsem, osem):
    nc = 2
    wid = lax.axis_index("s") * nc + lax.axis_index("c")
    rows_per_w = (t_total * 4 * 8) // _NW
    units_per_w = rows_per_w // 64
    u0 = wid * units_per_w
    m = u0 // (t_total // 8)
    ts = (u0 % (t_total // 8)) * 8
    pltpu.sync_copy(gt.at[m, :, pl.ds(ts, units_per_w * 8)], gate_v)
    it16 = lax.broadcasted_iota(jnp.int32, (16,), 0)
    pat_lo = lax.bitwise_and(it16, 7)
    pat_hi = pat_lo + 8
    jpat = lax.shift_right_logical(it16, 3) * 8
    niter = rows_per_w // _CH

    def fire_gather(i):
        buf = i % 6
        slot, half = (i // 2) % 3, i % 2
        toff = i * 16
        best = gate_v[0, pl.ds(toff, 16)]
        besti = jnp.zeros((16,), jnp.int32)
        for ee in range(1, 8):
            ge = gate_v[ee, pl.ds(toff, 16)]
            gtr = ge > best
            besti = jnp.where(gtr, ee, besti)
            best = jnp.where(gtr, ge, best)
        v16 = m * (t_total * 64) + (ts + toff + it16) * 64 + besti
        dup_lo = _vgather16(v16, pat_lo)
        dup_hi = _vgather16(v16, pat_hi)
        for k in range(8):
            vdup = dup_lo if k < 4 else dup_hi
            idx_v[buf, pl.ds(16 * k, 16)] = vdup + (16 * (k % 4) + jpat)
        dst = data_v.at[slot, pl.ds(half * _CH, _CH)]
        pltpu.async_copy(z2.at[idx_v.at[buf]], dst, gsem)

    def wait_gather(i):
        buf = i % 6
        slot, half = (i // 2) % 3, i % 2
        dst = data_v.at[slot, pl.ds(half * _CH, _CH)]
        pltpu.make_async_copy(z2.at[idx_v.at[buf]], dst, gsem).wait()

    def fire_out(p):
        qb = wid * rows_per_w + p * 2 * _CH
        pltpu.async_copy(data_v.at[p % 3], out.at[pl.ds(qb, 2 * _CH)], osem)

    def wait_out(p):
        qb = wid * rows_per_w + p * 2 * _CH
        pltpu.make_async_copy(data_v.at[p % 3], out.at[pl.ds(qb, 2 * _CH)], osem).wait()

    npair = niter // 2
    for g in range(4):
        fire_gather(g)
    for p in range(npair):
        if p + 2 < npair:
            if p >= 1:
                wait_out(p - 1)
            fire_gather(2 * (p + 2))
            fire_gather(2 * (p + 2) + 1)
        wait_gather(2 * p)
        wait_gather(2 * p + 1)
        fire_out(p)
    wait_out(npair - 2)
    wait_out(npair - 1)


def kernel(experts, gate):
    b, t, d, e = experts.shape
    nj = d // 128
    z2 = (experts.transpose(0, 1, 3, 2)
          .reshape(b, t, e, nj, 128)
          .transpose(0, 1, 3, 2, 4)
          .reshape(b * t * nj * e, 128))
    gt = gate.transpose(0, 2, 1)
    mesh = plsc.VectorSubcoreMesh(core_axis_name="c", subcore_axis_name="s")
    run = functools.partial(
        pl.kernel,
        mesh=mesh,
        out_type=jax.ShapeDtypeStruct((b * t * nj, 128), jnp.float32),
        scratch_types=[
            pltpu.VMEM((6, _CH), jnp.int32),
            pltpu.VMEM((3, 2 * _CH, 128), jnp.float32),
            pltpu.VMEM((8, 256), jnp.float32),
            pltpu.SemaphoreType.DMA,
            pltpu.SemaphoreType.DMA,
        ],
    )(functools.partial(_sc_body, t))
    out2 = run(z2, gt)
    return (out2.reshape(b, t // 8, nj, 8, 128)
            .transpose(0, 1, 3, 2, 4)
            .reshape(b, t, d))
